# Initial kernel scaffold; baseline (speedup 1.0000x reference)
import functools

import jax
import jax.numpy as jnp
from jax import lax
from jax.experimental import pallas as pl
from jax.experimental.pallas import tpu as pltpu
from jax.experimental.pallas import tpu_sc as plsc

B = 16384
NC, NS = 2, 16
NW = NC * NS
BPW = B // NW

D0, D1, D2 = 64, 32, 16

QC_RAW, QC_PAD = 33, 40
PC_RAW, PC_PAD = 26, 32
QIN_PAD = D0 + D1 + D2 + QC_PAD
PIN_PAD = D0 + D1 + PC_PAD

MID = 256
LAT = 64

BLK = 1024


def _sc_gather_body(q0, q1, q2, p0, p1, tq0, tq1, tq2, tp0, tp1,
                    oq0, oq1, oq2, op0, op1,
                    i0, i1, i2, i3, i4, r0, r1, r2, r3, r4, sem):
    wid = lax.axis_index("s") * NC + lax.axis_index("c")
    base = wid * BPW
    sl = pl.ds(base, BPW)
    pltpu.sync_copy(q0.at[sl], i0)
    pltpu.sync_copy(q1.at[sl], i1)
    pltpu.sync_copy(q2.at[sl], i2)
    pltpu.sync_copy(p0.at[sl], i3)
    pltpu.sync_copy(p1.at[sl], i4)
    c0 = pltpu.async_copy(tq0.at[i0], r0, sem)
    c1 = pltpu.async_copy(tq1.at[i1], r1, sem)
    c2 = pltpu.async_copy(tq2.at[i2], r2, sem)
    c3 = pltpu.async_copy(tp0.at[i3], r3, sem)
    c4 = pltpu.async_copy(tp1.at[i4], r4, sem)
    c0.wait()
    c1.wait()
    c2.wait()
    c3.wait()
    c4.wait()
    pltpu.sync_copy(r0, oq0.at[sl])
    pltpu.sync_copy(r1, oq1.at[sl])
    pltpu.sync_copy(r2, oq2.at[sl])
    pltpu.sync_copy(r3, op0.at[sl])
    pltpu.sync_copy(r4, op1.at[sl])


_sc_gather = functools.partial(
    pl.kernel,
    mesh=plsc.VectorSubcoreMesh(core_axis_name="c", subcore_axis_name="s"),
    out_type=[
        jax.ShapeDtypeStruct((B, D0), jnp.float32),
        jax.ShapeDtypeStruct((B, D1), jnp.float32),
        jax.ShapeDtypeStruct((B, D2), jnp.float32),
        jax.ShapeDtypeStruct((B, D0), jnp.float32),
        jax.ShapeDtypeStruct((B, D1), jnp.float32),
    ],
    scratch_types=[
        pltpu.VMEM((BPW,), jnp.int32),
        pltpu.VMEM((BPW,), jnp.int32),
        pltpu.VMEM((BPW,), jnp.int32),
        pltpu.VMEM((BPW,), jnp.int32),
        pltpu.VMEM((BPW,), jnp.int32),
        pltpu.VMEM((BPW, D0), jnp.float32),
        pltpu.VMEM((BPW, D1), jnp.float32),
        pltpu.VMEM((BPW, D2), jnp.float32),
        pltpu.VMEM((BPW, D0), jnp.float32),
        pltpu.VMEM((BPW, D1), jnp.float32),
        pltpu.SemaphoreType.DMA,
    ],
)(_sc_gather_body)


def _tc_body(eq0, eq1, eq2, qc, ep0, ep1, pc,
             wq1, bq1, wq2, bq2, wp1, bp1, wp2, bp2, out_ref):
    f32 = jnp.float32
    hq = (jnp.dot(eq0[...], wq1[0:D0, :], preferred_element_type=f32)
          + jnp.dot(eq1[...], wq1[D0:D0 + D1, :], preferred_element_type=f32)
          + jnp.dot(eq2[...], wq1[D0 + D1:D0 + D1 + D2, :], preferred_element_type=f32)
          + jnp.dot(qc[...], wq1[D0 + D1 + D2:QIN_PAD, :], preferred_element_type=f32)
          + bq1[...])
    hq = jnp.tanh(hq)
    qe = jnp.dot(hq, wq2[...], preferred_element_type=f32) + bq2[...]
    hp = (jnp.dot(ep0[...], wp1[0:D0, :], preferred_element_type=f32)
          + jnp.dot(ep1[...], wp1[D0:D0 + D1, :], preferred_element_type=f32)
          + jnp.dot(pc[...], wp1[D0 + D1:PIN_PAD, :], preferred_element_type=f32)
          + bp1[...])
    hp = jnp.tanh(hp)
    pe = jnp.dot(hp, wp2[...], preferred_element_type=f32) + bp2[...]
    d = jnp.sum(jnp.square(qe - pe), axis=1, keepdims=True)
    out_ref[...] = jnp.exp(-d)


def kernel(que, pro, que_tab0, que_tab1, que_tab2, pro_tab0, pro_tab1,
           que_W1, que_b1, que_W2, que_b2, pro_W1, pro_b1, pro_W2, pro_b2):
    f32 = jnp.float32
    qi0 = que[:, 0].astype(jnp.int32)
    qi1 = que[:, 1].astype(jnp.int32)
    qi2 = que[:, 2].astype(jnp.int32)
    pi0 = pro[:, 0].astype(jnp.int32)
    pi1 = pro[:, 1].astype(jnp.int32)
    qc = jnp.pad(que[:, 3:3 + QC_RAW], ((0, 0), (0, QC_PAD - QC_RAW)))
    pc = jnp.pad(pro[:, 2:2 + PC_RAW], ((0, 0), (0, PC_PAD - PC_RAW)))
    wq1 = jnp.pad(que_W1, ((0, QIN_PAD - que_W1.shape[0]), (0, 0)))
    wp1 = jnp.pad(pro_W1, ((0, PIN_PAD - pro_W1.shape[0]), (0, 0)))
    bq1 = que_b1.reshape(1, MID)
    bq2 = que_b2.reshape(1, LAT)
    bp1 = pro_b1.reshape(1, MID)
    bp2 = pro_b2.reshape(1, LAT)

    eq0, eq1, eq2, ep0, ep1 = _sc_gather(
        qi0, qi1, qi2, pi0, pi1, que_tab0, que_tab1, que_tab2, pro_tab0, pro_tab1)

    grid = (B // BLK,)
    row = lambda i: (i, 0)
    rep = lambda i: (0, 0)
    out = pl.pallas_call(
        _tc_body,
        grid=grid,
        in_specs=[
            pl.BlockSpec((BLK, D0), row),
            pl.BlockSpec((BLK, D1), row),
            pl.BlockSpec((BLK, D2), row),
            pl.BlockSpec((BLK, QC_PAD), row),
            pl.BlockSpec((BLK, D0), row),
            pl.BlockSpec((BLK, D1), row),
            pl.BlockSpec((BLK, PC_PAD), row),
            pl.BlockSpec((QIN_PAD, MID), rep),
            pl.BlockSpec((1, MID), rep),
            pl.BlockSpec((MID, LAT), rep),
            pl.BlockSpec((1, LAT), rep),
            pl.BlockSpec((PIN_PAD, MID), rep),
            pl.BlockSpec((1, MID), rep),
            pl.BlockSpec((MID, LAT), rep),
            pl.BlockSpec((1, LAT), rep),
        ],
        out_specs=pl.BlockSpec((BLK, 1), row),
        out_shape=jax.ShapeDtypeStruct((B, 1), f32),
    )(eq0, eq1, eq2, qc, ep0, ep1, pc,
      wq1, bq1, wq2, bq2, wp1, bp1, wp2, bp2)
    return out


# speedup vs baseline: 2.4958x; 2.4958x over previous
import functools

import jax
import jax.numpy as jnp
from jax import lax
from jax.experimental import pallas as pl
from jax.experimental.pallas import tpu as pltpu
from jax.experimental.pallas import tpu_sc as plsc

B = 16384
NC, NS = 2, 16
NW = NC * NS
BPW = B // NW

D0, D1, D2 = 64, 32, 16

QUE_DIM, PRO_DIM = 40, 30
QC_RAW = 33
PC_RAW = 26
QIN = D0 + D1 + D2 + QC_RAW
PIN = D0 + D1 + PC_RAW

MID = 256
LAT = 64

BLK = 1024


V0 = 100000
DPW = 4
HALF = B // 2


def _sc_big_body(q0, p0, tq0t, tp0t, oq0t, op0t, idx_v, row_v, out_v):
    wid = lax.axis_index("s") * NC + lax.axis_index("c")
    is_que = wid < NS
    dbase = jnp.where(is_que, wid, wid - NS) * DPW

    def side(idx_hbm, tabt, outt):
        pltpu.sync_copy(idx_hbm, idx_v)
        for k in range(DPW):
            d = dbase + k
            pltpu.sync_copy(tabt.at[d], row_v)
            for h in range(2):
                @pl.loop(0, HALF // 16)
                def _(j):
                    iv = idx_v[pl.ds(h * HALF + j * 16, 16)]
                    out_v[pl.ds(j * 16, 16)] = plsc.load_gather(row_v, [iv])
                pltpu.sync_copy(out_v, outt.at[d, pl.ds(h * HALF, HALF)])

    @pl.when(is_que)
    def _():
        side(q0, tq0t, oq0t)

    @pl.when(jnp.logical_not(is_que))
    def _():
        side(p0, tp0t, op0t)


@functools.lru_cache(maxsize=None)
def _get_sc_big():
    return functools.partial(
        pl.kernel,
        mesh=plsc.VectorSubcoreMesh(core_axis_name="c", subcore_axis_name="s"),
        compiler_params=pltpu.CompilerParams(needs_layout_passes=False),
        out_type=[
            jax.ShapeDtypeStruct((D0, B), jnp.float32),
            jax.ShapeDtypeStruct((D0, B), jnp.float32),
        ],
        scratch_types=[
            pltpu.VMEM((B,), jnp.int32),
            pltpu.VMEM((V0,), jnp.float32),
            pltpu.VMEM((HALF,), jnp.float32),
        ],
    )(_sc_big_body)


def _sc_small_body(q1, q2, p1, tq1, tq2, tp1,
                   oq1, oq2, op1,
                   i1, i2, i3, r1, r2, r3, sem):
    wid = lax.axis_index("s") * NC + lax.axis_index("c")
    base = wid * BPW
    sl = pl.ds(base, BPW)
    pltpu.sync_copy(q1.at[sl], i1)
    pltpu.sync_copy(q2.at[sl], i2)
    pltpu.sync_copy(p1.at[sl], i3)
    c1 = pltpu.async_copy(tq1.at[i1], r1, sem)
    c2 = pltpu.async_copy(tq2.at[i2], r2, sem)
    c3 = pltpu.async_copy(tp1.at[i3], r3, sem)
    c1.wait()
    c2.wait()
    c3.wait()
    pltpu.sync_copy(r1, oq1.at[sl])
    pltpu.sync_copy(r2, oq2.at[sl])
    pltpu.sync_copy(r3, op1.at[sl])


@functools.lru_cache(maxsize=None)
def _get_sc_small():
    return functools.partial(
        pl.kernel,
        mesh=plsc.VectorSubcoreMesh(core_axis_name="c", subcore_axis_name="s"),
        compiler_params=pltpu.CompilerParams(use_tc_tiling_on_sc=False,
                                             needs_layout_passes=False),
        out_type=[
        jax.ShapeDtypeStruct((B, D1), jnp.float32),
        jax.ShapeDtypeStruct((B, D2), jnp.float32),
        jax.ShapeDtypeStruct((B, D1), jnp.float32),
    ],
    scratch_types=[
        pltpu.VMEM((BPW,), jnp.int32),
        pltpu.VMEM((BPW,), jnp.int32),
        pltpu.VMEM((BPW,), jnp.int32),
        pltpu.VMEM((BPW, D1), jnp.float32),
        pltpu.VMEM((BPW, D2), jnp.float32),
        pltpu.VMEM((BPW, D1), jnp.float32),
        pltpu.SemaphoreType.DMA,
    ],
    )(_sc_small_body)


def _dot_t(lhs_t, rhs):
    return jax.lax.dot_general(lhs_t, rhs, (((0,), (0,)), ((), ())),
                               preferred_element_type=jnp.float32)


def _tc_body(eq0t, eq1, eq2, quet, ep0t, ep1, prot,
             wq1, wq1c, bq1, wq2, bq2, wp1, wp1c, bp1, wp2, bp2, out_ref):
    f32 = jnp.float32
    hq = (_dot_t(eq0t[...], wq1[0:D0, :])
          + jnp.dot(eq1[...], wq1[D0:D0 + D1, :], preferred_element_type=f32)
          + jnp.dot(eq2[...], wq1[D0 + D1:D0 + D1 + D2, :], preferred_element_type=f32)
          + _dot_t(quet[...], wq1c[...])
          + bq1[...])
    hq = jnp.tanh(hq)
    qe = jnp.dot(hq, wq2[...], preferred_element_type=f32) + bq2[...]
    hp = (_dot_t(ep0t[...], wp1[0:D0, :])
          + jnp.dot(ep1[...], wp1[D0:D0 + D1, :], preferred_element_type=f32)
          + _dot_t(prot[...], wp1c[...])
          + bp1[...])
    hp = jnp.tanh(hp)
    pe = jnp.dot(hp, wp2[...], preferred_element_type=f32) + bp2[...]
    d = jnp.sum(jnp.square(qe - pe), axis=1, keepdims=True)
    out_ref[...] = jnp.exp(-d)


def kernel(que, pro, que_tab0, que_tab1, que_tab2, pro_tab0, pro_tab1,
           que_W1, que_b1, que_W2, que_b2, pro_W1, pro_b1, pro_W2, pro_b2):
    f32 = jnp.float32
    qi0 = que[:, 0].astype(jnp.int32)
    qi1 = que[:, 1].astype(jnp.int32)
    qi2 = que[:, 2].astype(jnp.int32)
    pi0 = pro[:, 0].astype(jnp.int32)
    pi1 = pro[:, 1].astype(jnp.int32)
    quet = que.T
    prot = pro.T
    wq1c = jnp.zeros((QUE_DIM, MID), jnp.float32).at[3:3 + QC_RAW].set(
        que_W1[D0 + D1 + D2:])
    wp1c = jnp.zeros((PRO_DIM, MID), jnp.float32).at[2:2 + PC_RAW].set(
        pro_W1[D0 + D1:])
    wq1 = que_W1
    wp1 = pro_W1
    wq2 = que_W2
    wp2 = pro_W2
    bq1 = que_b1.reshape(1, MID)
    bq2 = que_b2.reshape(1, LAT)
    bp1 = pro_b1.reshape(1, MID)
    bp2 = pro_b2.reshape(1, LAT)

    eq0t, ep0t = _get_sc_big()(qi0, pi0, que_tab0.T, pro_tab0.T)
    eq1, eq2, ep1 = _get_sc_small()(qi1, qi2, pi1, que_tab1, que_tab2, pro_tab1)

    grid = (B // BLK,)
    row = lambda i: (i, 0)
    col = lambda i: (0, i)
    rep = lambda i: (0, 0)
    out = pl.pallas_call(
        _tc_body,
        grid=grid,
        in_specs=[
            pl.BlockSpec((D0, BLK), col),
            pl.BlockSpec((BLK, D1), row),
            pl.BlockSpec((BLK, D2), row),
            pl.BlockSpec((QUE_DIM, BLK), col),
            pl.BlockSpec((D0, BLK), col),
            pl.BlockSpec((BLK, D1), row),
            pl.BlockSpec((PRO_DIM, BLK), col),
            pl.BlockSpec((QIN, MID), rep),
            pl.BlockSpec((QUE_DIM, MID), rep),
            pl.BlockSpec((1, MID), rep),
            pl.BlockSpec((MID, LAT), rep),
            pl.BlockSpec((1, LAT), rep),
            pl.BlockSpec((PIN, MID), rep),
            pl.BlockSpec((PRO_DIM, MID), rep),
            pl.BlockSpec((1, MID), rep),
            pl.BlockSpec((MID, LAT), rep),
            pl.BlockSpec((1, LAT), rep),
        ],
        out_specs=pl.BlockSpec((BLK, 1), row),
        out_shape=jax.ShapeDtypeStruct((B, 1), f32),
    )(eq0t, eq1, eq2, quet, ep0t, ep1, prot,
      wq1, wq1c, bq1, wq2, bq2, wp1, wp1c, bp1, wp2, bp2)
    return out
